# baseline (speedup 1.0000x reference)
import functools

import jax
import jax.numpy as jnp
from jax import lax
from jax.experimental import pallas as pl
from jax.experimental.pallas import tpu as pltpu
from jax.experimental.pallas import tpu_sc as plsc

N = 50000
E = 800000
NC, NS = 2, 16
EPAD = 802816
CHUNK = 448
C2 = 1792
CC = 2000
EPT0 = E // NS
EPT = EPAD // NS
NP = 50176
RPT = NP // NS
CW = 8

_mesh = plsc.VectorSubcoreMesh(
    core_axis_name="c", subcore_axis_name="s", num_cores=NC, num_subcores=NS)
_sc_params = pltpu.CompilerParams(use_tc_tiling_on_sc=False)


def _gather_scatter_chunks(n_chunks, chunk, edge_base, src1d, dst1d, xtbl,
                           tbl, srcv, dstv, rows, semi, semg, sems):
    def idx_descs(k, b):
        e0 = edge_base + k * chunk
        return [
            pltpu.make_async_copy(src1d.at[pl.ds(e0, chunk)], srcv[b],
                                  semi[b]),
            pltpu.make_async_copy(dst1d.at[pl.ds(e0, chunk)], dstv[b],
                                  semi[b]),
        ]

    def gather_desc(b):
        return pltpu.make_async_copy(
            xtbl.at[srcv[b]], rows.at[pl.ds(b * chunk, chunk)], semg[b])

    for d in idx_descs(0, 0):
        d.start()
    for d in idx_descs(1, 1):
        d.start()
    for d in idx_descs(0, 0):
        d.wait()
    gather_desc(0).start()

    def pair(g, carry):
        for b in (0, 1):
            k = 2 * g + b
            nb = 1 - b

            @pl.when(k + 1 < n_chunks)
            def _():
                for d in idx_descs(k + 1, nb):
                    d.wait()
                gather_desc(nb).start()

            gather_desc(b).wait()
            pltpu.async_copy(rows.at[pl.ds(b * chunk, chunk)],
                             tbl.at[dstv[b]], sems[b], add=True).wait()

            @pl.when(k + 2 < n_chunks)
            def _():
                for d in idx_descs(k + 2, b):
                    d.start()
        return carry

    lax.fori_loop(0, n_chunks // 2, pair, 0)


def _seg_scratch(chunk, width):
    return [
        pltpu.VMEM_SHARED((NP, width), jnp.float32),
        pltpu.VMEM((chunk,), jnp.int32),
        pltpu.VMEM((chunk,), jnp.int32),
        pltpu.VMEM((chunk,), jnp.int32),
        pltpu.VMEM((chunk,), jnp.int32),
        pltpu.VMEM((2 * chunk, width), jnp.float32),
        pltpu.SemaphoreType.DMA,
        pltpu.SemaphoreType.DMA,
        pltpu.SemaphoreType.DMA,
        pltpu.SemaphoreType.DMA,
        pltpu.SemaphoreType.DMA,
        pltpu.SemaphoreType.DMA,
    ]


@functools.partial(
    pl.kernel,
    out_type=[
        jax.ShapeDtypeStruct((NP, CW), jnp.float32),
        jax.ShapeDtypeStruct((NP, CW), jnp.float32),
    ],
    mesh=_mesh,
    compiler_params=_sc_params,
    scratch_types=[
        pltpu.VMEM_SHARED((NP, CW), jnp.float32),
        pltpu.VMEM((CC,), jnp.int32),
        pltpu.VMEM((CC, CW), jnp.float32),
        pltpu.SemaphoreType.DMA,
    ],
)
def _sc_pass0(dstap1d, dstpp1d, zeros8, ones8, cnt8, deg8,
              tblc, dstv, onev, sems):
    c = lax.axis_index("c")
    s = lax.axis_index("s")
    pltpu.sync_copy(ones8, onev)
    pltpu.sync_copy(zeros8, tblc.at[pl.ds(s * RPT, RPT)])
    plsc.subcore_barrier()

    n_chunks = EPT0 // CC
    edge_base = s * EPT0

    def count_loop(d1d):
        def chunk(u, carry):
            e0 = edge_base + u * CC
            pltpu.sync_copy(d1d.at[pl.ds(e0, CC)], dstv)
            pltpu.async_copy(onev, tblc.at[dstv], sems, add=True).wait()
            return carry
        lax.fori_loop(0, n_chunks, chunk, 0)

    @pl.when(c == 0)
    def _():
        count_loop(dstap1d)

    @pl.when(c == 1)
    def _():
        count_loop(dstpp1d)

    plsc.subcore_barrier()

    @pl.when(c == 0)
    def _():
        pltpu.sync_copy(tblc.at[pl.ds(s * RPT, RPT)],
                        cnt8.at[pl.ds(s * RPT, RPT)])

    @pl.when(c == 1)
    def _():
        pltpu.sync_copy(tblc.at[pl.ds(s * RPT, RPT)],
                        deg8.at[pl.ds(s * RPT, RPT)])


@functools.partial(
    pl.kernel,
    out_type=[
        jax.ShapeDtypeStruct((NP, 32), jnp.float32),
        jax.ShapeDtypeStruct((NP, 32), jnp.float32),
    ],
    mesh=_mesh,
    compiler_params=_sc_params,
    scratch_types=_seg_scratch(CHUNK, 32),
)
def _sc_pass1(xa_lo, xa_hi, srcap1d, dstap1d, zeros32, agg_lo, agg_hi,
              tblf, sv0, sv1, dv0, dv1, rows, si0, si1, sg0, sg1, ss0, ss1):
    c = lax.axis_index("c")
    s = lax.axis_index("s")
    pltpu.sync_copy(zeros32, tblf.at[pl.ds(s * RPT, RPT)])
    plsc.subcore_barrier()

    n_chunks = EPT // CHUNK
    edge_base = s * EPT
    srcv, dstv = (sv0, sv1), (dv0, dv1)
    semi, semg, sems = (si0, si1), (sg0, sg1), (ss0, ss1)

    @pl.when(c == 0)
    def _():
        _gather_scatter_chunks(n_chunks, CHUNK, edge_base, srcap1d, dstap1d, xa_lo,
                               tblf, srcv, dstv, rows, semi, semg, sems)

    @pl.when(c == 1)
    def _():
        _gather_scatter_chunks(n_chunks, CHUNK, edge_base, srcap1d, dstap1d, xa_hi,
                               tblf, srcv, dstv, rows, semi, semg, sems)

    plsc.subcore_barrier()

    @pl.when(c == 0)
    def _():
        pltpu.sync_copy(tblf.at[pl.ds(s * RPT, RPT)],
                        agg_lo.at[pl.ds(s * RPT, RPT)])

    @pl.when(c == 1)
    def _():
        pltpu.sync_copy(tblf.at[pl.ds(s * RPT, RPT)],
                        agg_hi.at[pl.ds(s * RPT, RPT)])


@functools.partial(
    pl.kernel,
    out_type=[
        jax.ShapeDtypeStruct((NP, 16), jnp.float32),
        jax.ShapeDtypeStruct((NP, 16), jnp.float32),
    ],
    mesh=_mesh,
    compiler_params=_sc_params,
    scratch_types=_seg_scratch(C2, 16),
)
def _sc_pass2(y1a, y1b, srcpp1d, dstpp1d, zeros16, agg1_lo, agg1_hi,
              tblf, sv0, sv1, dv0, dv1, rows, si0, si1, sg0, sg1, ss0, ss1):
    c = lax.axis_index("c")
    s = lax.axis_index("s")
    pltpu.sync_copy(zeros16, tblf.at[pl.ds(s * RPT, RPT)])
    plsc.subcore_barrier()

    n_chunks = EPT // C2
    edge_base = s * EPT
    srcv, dstv = (sv0, sv1), (dv0, dv1)
    semi, semg, sems = (si0, si1), (sg0, sg1), (ss0, ss1)

    @pl.when(c == 0)
    def _():
        _gather_scatter_chunks(n_chunks, C2, edge_base, srcpp1d, dstpp1d,
                               y1a, tblf, srcv, dstv, rows, semi, semg, sems)

    @pl.when(c == 1)
    def _():
        _gather_scatter_chunks(n_chunks, C2, edge_base, srcpp1d, dstpp1d,
                               y1b, tblf, srcv, dstv, rows, semi, semg, sems)

    plsc.subcore_barrier()

    @pl.when(c == 0)
    def _():
        pltpu.sync_copy(tblf.at[pl.ds(s * RPT, RPT)],
                        agg1_lo.at[pl.ds(s * RPT, RPT)])

    @pl.when(c == 1)
    def _():
        pltpu.sync_copy(tblf.at[pl.ds(s * RPT, RPT)],
                        agg1_hi.at[pl.ds(s * RPT, RPT)])


@functools.partial(
    pl.kernel,
    out_type=[
        jax.ShapeDtypeStruct((NP, 32), jnp.float32),
        jax.ShapeDtypeStruct((NP, 32), jnp.float32),
    ],
    mesh=_mesh,
    compiler_params=_sc_params,
    scratch_types=_seg_scratch(CHUNK, 32),
)
def _sc_pass3(y2_lo, y2_hi, srcpp1d, dstpp1d, zeros32, agg2_lo, agg2_hi,
              tblf, sv0, sv1, dv0, dv1, rows, si0, si1, sg0, sg1, ss0, ss1):
    c = lax.axis_index("c")
    s = lax.axis_index("s")
    pltpu.sync_copy(zeros32, tblf.at[pl.ds(s * RPT, RPT)])
    plsc.subcore_barrier()

    n_chunks = EPT // CHUNK
    edge_base = s * EPT
    srcv, dstv = (sv0, sv1), (dv0, dv1)
    semi, semg, sems = (si0, si1), (sg0, sg1), (ss0, ss1)

    @pl.when(c == 0)
    def _():
        _gather_scatter_chunks(n_chunks, CHUNK, edge_base, srcpp1d, dstpp1d,
                               y2_lo, tblf, srcv, dstv, rows, semi, semg,
                               sems)

    @pl.when(c == 1)
    def _():
        _gather_scatter_chunks(n_chunks, CHUNK, edge_base, srcpp1d, dstpp1d,
                               y2_hi, tblf, srcv, dstv, rows, semi, semg,
                               sems)

    plsc.subcore_barrier()

    @pl.when(c == 0)
    def _():
        pltpu.sync_copy(tblf.at[pl.ds(s * RPT, RPT)],
                        agg2_lo.at[pl.ds(s * RPT, RPT)])

    @pl.when(c == 1)
    def _():
        pltpu.sync_copy(tblf.at[pl.ds(s * RPT, RPT)],
                        agg2_hi.at[pl.ds(s * RPT, RPT)])


BLK = 2000


def _rows_spec(cols):
    return pl.BlockSpec((BLK, cols), lambda i: (i, 0))


def _full_spec(r, cols):
    return pl.BlockSpec((r, cols), lambda i: (0, 0))


def _tc_a(deg_ref, cnt_ref, xp_ref, alo_ref, ahi_ref, wl1_ref, bl1_ref,
          wr1_ref, wl2_ref, bl2_ref, y1a_ref, y1b_ref, s1_ref, s2_ref):
    dis = lax.rsqrt(deg_ref[:, 0:1] + 1.0)
    xp = xp_ref[...]
    y1 = xp * dis
    y1a_ref[...] = y1[:, 0:16]
    y1b_ref[...] = y1[:, 16:32]
    inv = 1.0 / jnp.maximum(cnt_ref[:, 0:1], 1.0)
    mlo = alo_ref[...] * inv
    mhi = ahi_ref[...] * inv
    wl1 = wl1_ref[...]
    wl2 = wl2_ref[...]
    dot = functools.partial(jnp.dot, preferred_element_type=jnp.float32)
    s1_ref[...] = (dot(mlo, wl1[0:32, :]) + dot(mhi, wl1[32:64, :])
                   + bl1_ref[...] + dot(xp, wr1_ref[...]))
    s2_ref[...] = dot(mlo, wl2[0:32, :]) + dot(mhi, wl2[32:64, :]) + bl2_ref[...]


def _tc_b(a1lo_ref, a1hi_ref, y1a_ref, y1b_ref, deg_ref, s1_ref, w1_ref,
          b1_ref, wr2_ref, s2_ref, y2lo_ref, y2hi_ref, hp_ref):
    dis = lax.rsqrt(deg_ref[:, 0:1] + 1.0)
    glo = (a1lo_ref[...] + y1a_ref[...]) * dis
    ghi = (a1hi_ref[...] + y1b_ref[...]) * dis
    dot = functools.partial(jnp.dot, preferred_element_type=jnp.float32)
    w1 = w1_ref[...]
    h = jnp.maximum(dot(glo, w1[0:16, :]) + dot(ghi, w1[16:32, :])
                    + b1_ref[...] + s1_ref[...], 0.0)
    y2 = h * dis
    y2lo_ref[...] = y2[:, 0:32]
    y2hi_ref[...] = y2[:, 32:64]
    hp_ref[...] = dot(h, wr2_ref[...]) + s2_ref[...]


def _tc_c(a2lo_ref, a2hi_ref, y2lo_ref, y2hi_ref, deg_ref, hp_ref, w2_ref,
          b2_ref, out_ref):
    dis = lax.rsqrt(deg_ref[:, 0:1] + 1.0)
    dot = functools.partial(jnp.dot, preferred_element_type=jnp.float32)
    w2 = w2_ref[...]
    out_ref[...] = (dot((a2lo_ref[...] + y2lo_ref[...]) * dis, w2[0:32, :])
                    + dot((a2hi_ref[...] + y2hi_ref[...]) * dis, w2[32:64, :])
                    + b2_ref[...] + hp_ref[...])


def _pad_edges(src, dst):
    pad = EPAD - E
    src_p = jnp.concatenate([src, jnp.zeros((pad,), jnp.int32)])
    dst_p = jnp.concatenate([dst, jnp.full((pad,), N, jnp.int32)])
    return src_p, dst_p


def kernel(x_paper, x_author, edge_index_pp, edge_index_ap, W_gcn1, b_gcn1,
           Wl_sage1, bl_sage1, Wr_sage1, W_gcn2, b_gcn2, Wl_sage2, bl_sage2,
           Wr_sage2):
    f32 = jnp.float32
    srcpp1d, dstpp1d = _pad_edges(edge_index_pp[0], edge_index_pp[1])
    srcap1d, dstap1d = _pad_edges(edge_index_ap[0], edge_index_ap[1])
    xa_lo = x_author[:, 0:32]
    xa_hi = x_author[:, 32:64]
    zeros32 = jnp.zeros((RPT, 32), f32)
    zeros16 = jnp.zeros((RPT, 16), f32)
    zeros8 = jnp.zeros((RPT, CW), f32)
    ones8 = jnp.ones((CC, CW), f32)

    cnt8, deg8 = _sc_pass0(edge_index_ap[1], edge_index_pp[1], zeros8, ones8)
    agg_lo, agg_hi = _sc_pass1(xa_lo, xa_hi, srcap1d, dstap1d, zeros32)

    b2d = lambda b: b.reshape(1, 64)
    y1a, y1b, s1, s2m = pl.pallas_call(
        _tc_a,
        grid=(N // BLK,),
        in_specs=[_rows_spec(CW), _rows_spec(CW), _rows_spec(32),
                  _rows_spec(32), _rows_spec(32), _full_spec(64, 64),
                  _full_spec(1, 64), _full_spec(32, 64), _full_spec(64, 64),
                  _full_spec(1, 64)],
        out_specs=[_rows_spec(16), _rows_spec(16), _rows_spec(64),
                   _rows_spec(64)],
        out_shape=[jax.ShapeDtypeStruct((N, 16), f32),
                   jax.ShapeDtypeStruct((N, 16), f32),
                   jax.ShapeDtypeStruct((N, 64), f32),
                   jax.ShapeDtypeStruct((N, 64), f32)],
    )(deg8, cnt8, x_paper, agg_lo, agg_hi, Wl_sage1, b2d(bl_sage1),
      Wr_sage1, Wl_sage2, b2d(bl_sage2))

    agg1_lo, agg1_hi = _sc_pass2(y1a, y1b, srcpp1d, dstpp1d, zeros16)

    y2lo, y2hi, hpart = pl.pallas_call(
        _tc_b,
        grid=(N // BLK,),
        in_specs=[_rows_spec(16), _rows_spec(16), _rows_spec(16),
                  _rows_spec(16), _rows_spec(CW), _rows_spec(64),
                  _full_spec(32, 64), _full_spec(1, 64), _full_spec(64, 64),
                  _rows_spec(64)],
        out_specs=[_rows_spec(32), _rows_spec(32), _rows_spec(64)],
        out_shape=[jax.ShapeDtypeStruct((N, 32), f32),
                   jax.ShapeDtypeStruct((N, 32), f32),
                   jax.ShapeDtypeStruct((N, 64), f32)],
    )(agg1_lo, agg1_hi, y1a, y1b, deg8, s1, W_gcn1, b2d(b_gcn1), Wr_sage2,
      s2m)

    agg2_lo, agg2_hi = _sc_pass3(y2lo, y2hi, srcpp1d, dstpp1d, zeros32)

    out = pl.pallas_call(
        _tc_c,
        grid=(N // BLK,),
        in_specs=[_rows_spec(32), _rows_spec(32), _rows_spec(32),
                  _rows_spec(32), _rows_spec(CW), _rows_spec(64),
                  _full_spec(64, 64), _full_spec(1, 64)],
        out_specs=_rows_spec(64),
        out_shape=jax.ShapeDtypeStruct((N, 64), f32),
    )(agg2_lo, agg2_hi, y2lo, y2hi, deg8, hpart, W_gcn2, b2d(b_gcn2))

    return out


# speedup vs baseline: 1.0493x; 1.0493x over previous
import functools

import jax
import jax.numpy as jnp
from jax import lax
from jax.experimental import pallas as pl
from jax.experimental.pallas import tpu as pltpu
from jax.experimental.pallas import tpu_sc as plsc

N = 50000
E = 800000
NC, NS = 2, 16
EPAD = 802816
CHUNK = 448
C2 = 1792
CC = 3136
EPT = EPAD // NS
NP = 50176
RPT = NP // NS
CW = 8

_mesh = plsc.VectorSubcoreMesh(
    core_axis_name="c", subcore_axis_name="s", num_cores=NC, num_subcores=NS)
_sc_params = pltpu.CompilerParams(use_tc_tiling_on_sc=False)


def _gather_scatter_chunks(n_chunks, chunk, edge_base, src1d, dst1d, xtbl,
                           tbl, srcv, dstv, rows, semi, semd, semg, sems):
    def src_desc(k, b):
        return pltpu.make_async_copy(
            src1d.at[pl.ds(edge_base + k * chunk, chunk)], srcv[b], semi[b])

    def dst_desc(k, b):
        return pltpu.make_async_copy(
            dst1d.at[pl.ds(edge_base + k * chunk, chunk)], dstv[b], semd[b])

    def gather_desc(b):
        return pltpu.make_async_copy(
            xtbl.at[srcv[b]], rows.at[pl.ds(b * chunk, chunk)], semg[b])

    def scatter_drain(b):
        pltpu.make_async_copy(xtbl.at[pl.ds(0, chunk)],
                              rows.at[pl.ds(b * chunk, chunk)],
                              sems[b]).wait()

    src_desc(0, 0).start()
    src_desc(1, 1).start()
    dst_desc(0, 0).start()
    src_desc(0, 0).wait()
    gather_desc(0).start()

    def pair(g, carry):
        for b in (0, 1):
            k = 2 * g + b
            nb = 1 - b

            @pl.when(k >= 1)
            def _():
                scatter_drain(nb)

            @pl.when(k + 1 < n_chunks)
            def _():
                src_desc(k + 1, nb).wait()
                gather_desc(nb).start()
                dst_desc(k + 1, nb).start()

            gather_desc(b).wait()
            dst_desc(k, b).wait()
            pltpu.async_copy(rows.at[pl.ds(b * chunk, chunk)],
                             tbl.at[dstv[b]], sems[b], add=True)

            @pl.when(k + 2 < n_chunks)
            def _():
                src_desc(k + 2, b).start()
        return carry

    lax.fori_loop(0, n_chunks // 2, pair, 0)
    scatter_drain((n_chunks - 1) % 2)


def _seg_scratch(chunk, width):
    return [
        pltpu.VMEM_SHARED((NP, width), jnp.float32),
        pltpu.VMEM((chunk,), jnp.int32),
        pltpu.VMEM((chunk,), jnp.int32),
        pltpu.VMEM((chunk,), jnp.int32),
        pltpu.VMEM((chunk,), jnp.int32),
        pltpu.VMEM((2 * chunk, width), jnp.float32),
        pltpu.SemaphoreType.DMA,
        pltpu.SemaphoreType.DMA,
        pltpu.SemaphoreType.DMA,
        pltpu.SemaphoreType.DMA,
        pltpu.SemaphoreType.DMA,
        pltpu.SemaphoreType.DMA,
        pltpu.SemaphoreType.DMA,
        pltpu.SemaphoreType.DMA,
    ]


@functools.partial(
    pl.kernel,
    out_type=[
        jax.ShapeDtypeStruct((NP, CW), jnp.float32),
        jax.ShapeDtypeStruct((NP, CW), jnp.float32),
    ],
    mesh=_mesh,
    compiler_params=_sc_params,
    scratch_types=[
        pltpu.VMEM_SHARED((NP, CW), jnp.float32),
        pltpu.VMEM((CC,), jnp.int32),
        pltpu.VMEM((CC, CW), jnp.float32),
        pltpu.SemaphoreType.DMA,
    ],
)
def _sc_pass0(dstap1d, dstpp1d, zeros8, ones8, cnt8, deg8,
              tblc, dstv, onev, sems):
    c = lax.axis_index("c")
    s = lax.axis_index("s")
    pltpu.sync_copy(ones8, onev)
    pltpu.sync_copy(zeros8, tblc.at[pl.ds(s * RPT, RPT)])
    plsc.subcore_barrier()

    n_chunks = EPT // CC
    edge_base = s * EPT

    def count_loop(d1d):
        def chunk(u, carry):
            e0 = edge_base + u * CC
            pltpu.sync_copy(d1d.at[pl.ds(e0, CC)], dstv)
            pltpu.async_copy(onev, tblc.at[dstv], sems, add=True).wait()
            return carry
        lax.fori_loop(0, n_chunks, chunk, 0)

    @pl.when(c == 0)
    def _():
        count_loop(dstap1d)

    @pl.when(c == 1)
    def _():
        count_loop(dstpp1d)

    plsc.subcore_barrier()

    @pl.when(c == 0)
    def _():
        pltpu.sync_copy(tblc.at[pl.ds(s * RPT, RPT)],
                        cnt8.at[pl.ds(s * RPT, RPT)])

    @pl.when(c == 1)
    def _():
        pltpu.sync_copy(tblc.at[pl.ds(s * RPT, RPT)],
                        deg8.at[pl.ds(s * RPT, RPT)])


@functools.partial(
    pl.kernel,
    out_type=[
        jax.ShapeDtypeStruct((NP, 32), jnp.float32),
        jax.ShapeDtypeStruct((NP, 32), jnp.float32),
    ],
    mesh=_mesh,
    compiler_params=_sc_params,
    scratch_types=_seg_scratch(CHUNK, 32),
)
def _sc_pass1(xa_lo, xa_hi, srcap1d, dstap1d, zeros32, agg_lo, agg_hi,
              tblf, sv0, sv1, dv0, dv1, rows, si0, si1, sd0, sd1, sg0, sg1, ss0, ss1):
    c = lax.axis_index("c")
    s = lax.axis_index("s")
    pltpu.sync_copy(zeros32, tblf.at[pl.ds(s * RPT, RPT)])
    plsc.subcore_barrier()

    n_chunks = EPT // CHUNK
    edge_base = s * EPT
    srcv, dstv = (sv0, sv1), (dv0, dv1)
    semi, semd = (si0, si1), (sd0, sd1)
    semg, sems = (sg0, sg1), (ss0, ss1)

    @pl.when(c == 0)
    def _():
        _gather_scatter_chunks(n_chunks, CHUNK, edge_base, srcap1d, dstap1d, xa_lo,
                               tblf, srcv, dstv, rows, semi, semd, semg, sems)

    @pl.when(c == 1)
    def _():
        _gather_scatter_chunks(n_chunks, CHUNK, edge_base, srcap1d, dstap1d, xa_hi,
                               tblf, srcv, dstv, rows, semi, semd, semg, sems)

    plsc.subcore_barrier()

    @pl.when(c == 0)
    def _():
        pltpu.sync_copy(tblf.at[pl.ds(s * RPT, RPT)],
                        agg_lo.at[pl.ds(s * RPT, RPT)])

    @pl.when(c == 1)
    def _():
        pltpu.sync_copy(tblf.at[pl.ds(s * RPT, RPT)],
                        agg_hi.at[pl.ds(s * RPT, RPT)])


@functools.partial(
    pl.kernel,
    out_type=[
        jax.ShapeDtypeStruct((NP, 16), jnp.float32),
        jax.ShapeDtypeStruct((NP, 16), jnp.float32),
    ],
    mesh=_mesh,
    compiler_params=_sc_params,
    scratch_types=_seg_scratch(C2, 16),
)
def _sc_pass2(y1a, y1b, srcpp1d, dstpp1d, zeros16, agg1_lo, agg1_hi,
              tblf, sv0, sv1, dv0, dv1, rows, si0, si1, sd0, sd1, sg0, sg1, ss0, ss1):
    c = lax.axis_index("c")
    s = lax.axis_index("s")
    pltpu.sync_copy(zeros16, tblf.at[pl.ds(s * RPT, RPT)])
    plsc.subcore_barrier()

    n_chunks = EPT // C2
    edge_base = s * EPT
    srcv, dstv = (sv0, sv1), (dv0, dv1)
    semi, semd = (si0, si1), (sd0, sd1)
    semg, sems = (sg0, sg1), (ss0, ss1)

    @pl.when(c == 0)
    def _():
        _gather_scatter_chunks(n_chunks, C2, edge_base, srcpp1d, dstpp1d,
                               y1a, tblf, srcv, dstv, rows, semi, semd, semg, sems)

    @pl.when(c == 1)
    def _():
        _gather_scatter_chunks(n_chunks, C2, edge_base, srcpp1d, dstpp1d,
                               y1b, tblf, srcv, dstv, rows, semi, semd, semg, sems)

    plsc.subcore_barrier()

    @pl.when(c == 0)
    def _():
        pltpu.sync_copy(tblf.at[pl.ds(s * RPT, RPT)],
                        agg1_lo.at[pl.ds(s * RPT, RPT)])

    @pl.when(c == 1)
    def _():
        pltpu.sync_copy(tblf.at[pl.ds(s * RPT, RPT)],
                        agg1_hi.at[pl.ds(s * RPT, RPT)])


@functools.partial(
    pl.kernel,
    out_type=[
        jax.ShapeDtypeStruct((NP, 32), jnp.float32),
        jax.ShapeDtypeStruct((NP, 32), jnp.float32),
    ],
    mesh=_mesh,
    compiler_params=_sc_params,
    scratch_types=_seg_scratch(CHUNK, 32),
)
def _sc_pass3(y2_lo, y2_hi, srcpp1d, dstpp1d, zeros32, agg2_lo, agg2_hi,
              tblf, sv0, sv1, dv0, dv1, rows, si0, si1, sd0, sd1, sg0, sg1, ss0, ss1):
    c = lax.axis_index("c")
    s = lax.axis_index("s")
    pltpu.sync_copy(zeros32, tblf.at[pl.ds(s * RPT, RPT)])
    plsc.subcore_barrier()

    n_chunks = EPT // CHUNK
    edge_base = s * EPT
    srcv, dstv = (sv0, sv1), (dv0, dv1)
    semi, semd = (si0, si1), (sd0, sd1)
    semg, sems = (sg0, sg1), (ss0, ss1)

    @pl.when(c == 0)
    def _():
        _gather_scatter_chunks(n_chunks, CHUNK, edge_base, srcpp1d, dstpp1d,
                               y2_lo, tblf, srcv, dstv, rows, semi, semd,
                               semg, sems)

    @pl.when(c == 1)
    def _():
        _gather_scatter_chunks(n_chunks, CHUNK, edge_base, srcpp1d, dstpp1d,
                               y2_hi, tblf, srcv, dstv, rows, semi, semd,
                               semg, sems)

    plsc.subcore_barrier()

    @pl.when(c == 0)
    def _():
        pltpu.sync_copy(tblf.at[pl.ds(s * RPT, RPT)],
                        agg2_lo.at[pl.ds(s * RPT, RPT)])

    @pl.when(c == 1)
    def _():
        pltpu.sync_copy(tblf.at[pl.ds(s * RPT, RPT)],
                        agg2_hi.at[pl.ds(s * RPT, RPT)])


BLK = 2000


def _rows_spec(cols):
    return pl.BlockSpec((BLK, cols), lambda i: (i, 0))


def _full_spec(r, cols):
    return pl.BlockSpec((r, cols), lambda i: (0, 0))


def _tc_a(deg_ref, cnt_ref, xp_ref, alo_ref, ahi_ref, wl1_ref, bl1_ref,
          wr1_ref, wl2_ref, bl2_ref, y1a_ref, y1b_ref, s1_ref, s2_ref):
    dis = lax.rsqrt(deg_ref[:, 0:1] + 1.0)
    xp = xp_ref[...]
    y1 = xp * dis
    y1a_ref[...] = y1[:, 0:16]
    y1b_ref[...] = y1[:, 16:32]
    inv = 1.0 / jnp.maximum(cnt_ref[:, 0:1], 1.0)
    mlo = alo_ref[...] * inv
    mhi = ahi_ref[...] * inv
    wl1 = wl1_ref[...]
    wl2 = wl2_ref[...]
    dot = functools.partial(jnp.dot, preferred_element_type=jnp.float32)
    s1_ref[...] = (dot(mlo, wl1[0:32, :]) + dot(mhi, wl1[32:64, :])
                   + bl1_ref[...] + dot(xp, wr1_ref[...]))
    s2_ref[...] = dot(mlo, wl2[0:32, :]) + dot(mhi, wl2[32:64, :]) + bl2_ref[...]


def _tc_b(a1lo_ref, a1hi_ref, y1a_ref, y1b_ref, deg_ref, s1_ref, w1_ref,
          b1_ref, wr2_ref, s2_ref, y2lo_ref, y2hi_ref, hp_ref):
    dis = lax.rsqrt(deg_ref[:, 0:1] + 1.0)
    glo = (a1lo_ref[...] + y1a_ref[...]) * dis
    ghi = (a1hi_ref[...] + y1b_ref[...]) * dis
    dot = functools.partial(jnp.dot, preferred_element_type=jnp.float32)
    w1 = w1_ref[...]
    h = jnp.maximum(dot(glo, w1[0:16, :]) + dot(ghi, w1[16:32, :])
                    + b1_ref[...] + s1_ref[...], 0.0)
    y2 = h * dis
    y2lo_ref[...] = y2[:, 0:32]
    y2hi_ref[...] = y2[:, 32:64]
    hp_ref[...] = dot(h, wr2_ref[...]) + s2_ref[...]


def _tc_c(a2lo_ref, a2hi_ref, y2lo_ref, y2hi_ref, deg_ref, hp_ref, w2_ref,
          b2_ref, out_ref):
    dis = lax.rsqrt(deg_ref[:, 0:1] + 1.0)
    dot = functools.partial(jnp.dot, preferred_element_type=jnp.float32)
    w2 = w2_ref[...]
    out_ref[...] = (dot((a2lo_ref[...] + y2lo_ref[...]) * dis, w2[0:32, :])
                    + dot((a2hi_ref[...] + y2hi_ref[...]) * dis, w2[32:64, :])
                    + b2_ref[...] + hp_ref[...])


def _pad_edges(src, dst):
    pad = EPAD - E
    src_p = jnp.concatenate([src, jnp.zeros((pad,), jnp.int32)])
    dst_p = jnp.concatenate([dst, jnp.full((pad,), N, jnp.int32)])
    return src_p, dst_p


def kernel(x_paper, x_author, edge_index_pp, edge_index_ap, W_gcn1, b_gcn1,
           Wl_sage1, bl_sage1, Wr_sage1, W_gcn2, b_gcn2, Wl_sage2, bl_sage2,
           Wr_sage2):
    f32 = jnp.float32
    srcpp1d, dstpp1d = _pad_edges(edge_index_pp[0], edge_index_pp[1])
    srcap1d, dstap1d = _pad_edges(edge_index_ap[0], edge_index_ap[1])
    xa_lo = x_author[:, 0:32]
    xa_hi = x_author[:, 32:64]
    zeros32 = jnp.zeros((RPT, 32), f32)
    zeros16 = jnp.zeros((RPT, 16), f32)
    zeros8 = jnp.zeros((RPT, CW), f32)
    ones8 = jnp.ones((CC, CW), f32)

    cnt8, deg8 = _sc_pass0(dstap1d, dstpp1d, zeros8, ones8)
    agg_lo, agg_hi = _sc_pass1(xa_lo, xa_hi, srcap1d, dstap1d, zeros32)

    b2d = lambda b: b.reshape(1, 64)
    y1a, y1b, s1, s2m = pl.pallas_call(
        _tc_a,
        grid=(N // BLK,),
        in_specs=[_rows_spec(CW), _rows_spec(CW), _rows_spec(32),
                  _rows_spec(32), _rows_spec(32), _full_spec(64, 64),
                  _full_spec(1, 64), _full_spec(32, 64), _full_spec(64, 64),
                  _full_spec(1, 64)],
        out_specs=[_rows_spec(16), _rows_spec(16), _rows_spec(64),
                   _rows_spec(64)],
        out_shape=[jax.ShapeDtypeStruct((N, 16), f32),
                   jax.ShapeDtypeStruct((N, 16), f32),
                   jax.ShapeDtypeStruct((N, 64), f32),
                   jax.ShapeDtypeStruct((N, 64), f32)],
    )(deg8, cnt8, x_paper, agg_lo, agg_hi, Wl_sage1, b2d(bl_sage1),
      Wr_sage1, Wl_sage2, b2d(bl_sage2))

    agg1_lo, agg1_hi = _sc_pass2(y1a, y1b, srcpp1d, dstpp1d, zeros16)

    y2lo, y2hi, hpart = pl.pallas_call(
        _tc_b,
        grid=(N // BLK,),
        in_specs=[_rows_spec(16), _rows_spec(16), _rows_spec(16),
                  _rows_spec(16), _rows_spec(CW), _rows_spec(64),
                  _full_spec(32, 64), _full_spec(1, 64), _full_spec(64, 64),
                  _rows_spec(64)],
        out_specs=[_rows_spec(32), _rows_spec(32), _rows_spec(64)],
        out_shape=[jax.ShapeDtypeStruct((N, 32), f32),
                   jax.ShapeDtypeStruct((N, 32), f32),
                   jax.ShapeDtypeStruct((N, 64), f32)],
    )(agg1_lo, agg1_hi, y1a, y1b, deg8, s1, W_gcn1, b2d(b_gcn1), Wr_sage2,
      s2m)

    agg2_lo, agg2_hi = _sc_pass3(y2lo, y2hi, srcpp1d, dstpp1d, zeros32)

    out = pl.pallas_call(
        _tc_c,
        grid=(N // BLK,),
        in_specs=[_rows_spec(32), _rows_spec(32), _rows_spec(32),
                  _rows_spec(32), _rows_spec(CW), _rows_spec(64),
                  _full_spec(64, 64), _full_spec(1, 64)],
        out_specs=_rows_spec(64),
        out_shape=jax.ShapeDtypeStruct((N, 64), f32),
    )(agg2_lo, agg2_hi, y2lo, y2hi, deg8, hpart, W_gcn2, b2d(b_gcn2))

    return out
